# baseline (speedup 1.0000x reference)
import functools

import jax
import jax.numpy as jnp
from jax import lax
from jax.experimental import pallas as pl
from jax.experimental.pallas import tpu as pltpu
from jax.experimental.pallas import tpu_sc as plsc

N = 10000
E = 320000
D = 128
ALPHA = 0.01
K = 10

NC = 2
NS = 16
NTILES = NC * NS
W = 64
NWIN = 159
NB = 3
NG = NWIN // NB
EPT = NWIN * W
PACK = 16384
E_PAD = NTILES * EPT
ROW_STRIDE = 624
ROW_CHUNK = 640

_mesh = plsc.VectorSubcoreMesh(
    core_axis_name="c", subcore_axis_name="s", num_cores=NC, num_subcores=NS)


def _mlp_body(x_ref, w1_ref, b1_ref, w2_ref, b2_ref, o_ref):
    h = jnp.dot(x_ref[...], w1_ref[...], preferred_element_type=jnp.float32)
    h = jnp.maximum(h + b1_ref[...], 0.0)
    o = jnp.dot(h, w2_ref[...], preferred_element_type=jnp.float32)
    o_ref[...] = jnp.maximum(o + b2_ref[...], 0.0)


def _mlp(x, W1, b1, W2, b2):
    return pl.pallas_call(
        _mlp_body,
        grid=(5,),
        in_specs=[
            pl.BlockSpec((2000, D), lambda i: (i, 0)),
            pl.BlockSpec((D, D), lambda i: (0, 0)),
            pl.BlockSpec((1, D), lambda i: (0, 0)),
            pl.BlockSpec((D, D), lambda i: (0, 0)),
            pl.BlockSpec((1, D), lambda i: (0, 0)),
        ],
        out_specs=pl.BlockSpec((2000, D), lambda i: (i, 0)),
        out_shape=jax.ShapeDtypeStruct((N, D), jnp.float32),
    )(x, W1, b1.reshape(1, D), W2, b2.reshape(1, D))


def _combine_body(p_ref, h_ref, o_ref):
    agg = p_ref[0] + p_ref[1]
    o_ref[...] = (1.0 - ALPHA) * agg + ALPHA * h_ref[...]


def _combine(parts, h):
    return pl.pallas_call(
        _combine_body,
        grid=(5,),
        in_specs=[
            pl.BlockSpec((2, 2000, D), lambda i: (0, i, 0)),
            pl.BlockSpec((2000, D), lambda i: (i, 0)),
        ],
        out_specs=pl.BlockSpec((2000, D), lambda i: (i, 0)),
        out_shape=jax.ShapeDtypeStruct((N, D), jnp.float32),
    )(parts, h)


@functools.partial(
    pl.kernel,
    out_type=jax.ShapeDtypeStruct((NC, N, D), jnp.float32),
    mesh=_mesh,
    scratch_types=[
        pltpu.VMEM_SHARED((N, D), jnp.float32),
        pltpu.VMEM((EPT,), jnp.int32),
        pltpu.VMEM((EPT // 2,), jnp.int32),
        pltpu.VMEM((2 * NB, W), jnp.int32),
        pltpu.VMEM((NB, W, D), jnp.float32),
        pltpu.SemaphoreType.DMA((NB,)),
        pltpu.SemaphoreType.DMA((NB,)),
    ],
    compiler_params=pltpu.CompilerParams(needs_layout_passes=False),
)
def _propagate(h_hbm, pk_hbm, wgt_hbm, out_hbm,
               agg, pk_t, wgt_t, idxb, gbuf, gsem, ssem):
    c = lax.axis_index("c")
    s = lax.axis_index("s")
    slab = c * NS + s

    pltpu.sync_copy(pk_hbm.at[slab, 0], pk_t)
    pltpu.sync_copy(wgt_hbm.at[slab, 0], wgt_t)

    def _zero_row(r, carry):
        for f in range(D // 16):
            gbuf[0, r, pl.ds(f * 16, 16)] = jnp.zeros((16,), jnp.float32)
        return carry
    lax.fori_loop(0, W, _zero_row, 0)
    base = s * ROW_STRIDE
    for t in range(ROW_CHUNK // W):
        pltpu.sync_copy(gbuf.at[0], agg.at[pl.ds(base + t * W, W)])
    plsc.subcore_barrier()

    def _unpack(j, b):
        for v in range(W // 16):
            pk = pk_t[pl.ds(j * W + v * 16, 16)]
            idxb[b, pl.ds(v * 16, 16)] = lax.shift_right_logical(pk, 14)
            idxb[NB + b, pl.ds(v * 16, 16)] = lax.bitwise_and(pk, PACK - 1)

    def _gather(j, b):
        _unpack(j, b)
        pltpu.async_copy(h_hbm.at[idxb.at[b]], gbuf.at[b], gsem.at[b])

    def _gather_wait(b):
        pltpu.make_async_copy(
            h_hbm.at[idxb.at[b]], gbuf.at[b], gsem.at[b]).wait()

    def _scatter(b):
        pltpu.async_copy(gbuf.at[b], agg.at[idxb.at[NB + b]], ssem.at[b],
                         add=True)

    def _scatter_wait(b):
        pltpu.make_async_copy(gbuf.at[b], agg.at[idxb.at[NB + b]],
                              ssem.at[b]).wait()

    _gather(0, 0)
    _gather(1, 1)

    mask_hi = jnp.full((16,), 0xFFFF0000, jnp.uint32)
    shift16 = jnp.full((16,), 16, jnp.uint32)

    def _wexpand(vec_i32):
        u32 = plsc.bitcast(vec_i32, jnp.uint32)
        lo = plsc.bitcast(lax.shift_left(u32, shift16), jnp.float32)
        hi = plsc.bitcast(lax.bitwise_and(u32, mask_hi), jnp.float32)
        return lo, hi

    def _group(g, carry):
        for b in range(NB):
            j = g * NB + b
            bf = (b + 2) % NB

            _gather_wait(b)

            if b == 0:
                @pl.when(g >= 1)
                def _():
                    _scatter_wait(bf)
                _gather(j + 2, bf)
            else:
                @pl.when(g < NG - 1)
                def _():
                    _scatter_wait(bf)
                    _gather(j + 2, bf)

            def _scale_blk(k, carry2):
                for pair in range(4):
                    e0 = k * 8 + pair * 2
                    pidx = jnp.full((16,), 0, jnp.int32) + (j * (W // 2)
                                                            + e0 // 2)
                    w_even, w_odd = _wexpand(plsc.load_gather(wgt_t, [pidx]))
                    for e, wv in ((e0, w_even), (e0 + 1, w_odd)):
                        for f in range(D // 16):
                            sl = pl.ds(f * 16, 16)
                            gbuf[b, e, sl] = gbuf[b, e, sl] * wv
                return carry2
            lax.fori_loop(0, W // 8, _scale_blk, 0)

            _scatter(b)
        return carry
    lax.fori_loop(0, NG, _group, 0)

    for b in range(NB):
        _scatter_wait(b)

    plsc.subcore_barrier()
    pltpu.sync_copy(agg.at[pl.ds(base, ROW_CHUNK)],
                    out_hbm.at[c, pl.ds(base, ROW_CHUNK)])


def kernel(x, edge_index, edge_weight, W1, b1, W2, b2):
    h = _mlp(x, W1, b1, W2, b2)

    pad = E_PAD - E
    fill = (jnp.arange(pad, dtype=jnp.int32) * 997) % N
    col = jnp.concatenate([edge_index[1], fill])
    row = jnp.concatenate([edge_index[0], fill])
    packed = (col * PACK + row).reshape(NTILES, 1, EPT)
    wf = jnp.concatenate([edge_weight, jnp.zeros((pad,), jnp.float32)])
    wu = lax.bitcast_convert_type(wf, jnp.uint32)
    wr = (wu + 0x7FFF + ((wu >> 16) & 1)) >> 16
    wpk = lax.bitcast_convert_type(
        wr[0::2] | (wr[1::2] << 16), jnp.int32).reshape(NTILES, 1, EPT // 2)

    for _ in range(K):
        parts = _propagate(h, packed, wpk)
        h = _combine(parts, h)
    return h


# speedup vs baseline: 2.0937x; 2.0937x over previous
import functools

import jax
import jax.numpy as jnp
from jax import lax
from jax.experimental import pallas as pl
from jax.experimental.pallas import tpu as pltpu
from jax.experimental.pallas import tpu_sc as plsc

N = 10000
E = 320000
D = 128
ALPHA = 0.01
K = 10

NC = 2
NS = 16
NTILES = NC * NS
W = 80
NWIN = 126
NB = 3
NG = NWIN // NB
EPT = NWIN * W
PACK = 16384
E_PAD = NTILES * EPT
ROW_STRIDE = 624
ROW_CHUNK = 640

_mesh = plsc.VectorSubcoreMesh(
    core_axis_name="c", subcore_axis_name="s", num_cores=NC, num_subcores=NS)


def _mlp_body(x_ref, w1_ref, b1_ref, w2_ref, b2_ref, o_ref):
    h = jnp.dot(x_ref[...], w1_ref[...], preferred_element_type=jnp.float32)
    h = jnp.maximum(h + b1_ref[...], 0.0)
    o = jnp.dot(h, w2_ref[...], preferred_element_type=jnp.float32)
    o_ref[...] = jnp.maximum(o + b2_ref[...], 0.0)


def _mlp(x, W1, b1, W2, b2):
    return pl.pallas_call(
        _mlp_body,
        grid=(5,),
        in_specs=[
            pl.BlockSpec((2000, D), lambda i: (i, 0)),
            pl.BlockSpec((D, D), lambda i: (0, 0)),
            pl.BlockSpec((1, D), lambda i: (0, 0)),
            pl.BlockSpec((D, D), lambda i: (0, 0)),
            pl.BlockSpec((1, D), lambda i: (0, 0)),
        ],
        out_specs=pl.BlockSpec((2000, D), lambda i: (i, 0)),
        out_shape=jax.ShapeDtypeStruct((N, D), jnp.float32),
    )(x, W1, b1.reshape(1, D), W2, b2.reshape(1, D))


def _combine_body(p_ref, h_ref, o_ref):
    agg = p_ref[0] + p_ref[1]
    o_ref[...] = (1.0 - ALPHA) * agg + ALPHA * h_ref[...]


def _combine(parts, h):
    return pl.pallas_call(
        _combine_body,
        grid=(5,),
        in_specs=[
            pl.BlockSpec((2, 2000, D), lambda i: (0, i, 0)),
            pl.BlockSpec((2000, D), lambda i: (i, 0)),
        ],
        out_specs=pl.BlockSpec((2000, D), lambda i: (i, 0)),
        out_shape=jax.ShapeDtypeStruct((N, D), jnp.float32),
    )(parts, h)


@functools.partial(
    pl.kernel,
    out_type=jax.ShapeDtypeStruct((NC, N, D), jnp.float32),
    mesh=_mesh,
    scratch_types=[
        pltpu.VMEM_SHARED((N, D), jnp.float32),
        pltpu.VMEM((EPT,), jnp.int32),
        pltpu.VMEM((EPT // 2,), jnp.int32),
        pltpu.VMEM((2 * NB, W), jnp.int32),
        pltpu.VMEM((NB, W, D), jnp.float32),
        pltpu.SemaphoreType.DMA((NB,)),
        pltpu.SemaphoreType.DMA((NB,)),
    ],
    compiler_params=pltpu.CompilerParams(needs_layout_passes=False),
)
def _propagate(h_hbm, pk_hbm, wgt_hbm, out_hbm,
               agg, pk_t, wgt_t, idxb, gbuf, gsem, ssem):
    c = lax.axis_index("c")
    s = lax.axis_index("s")
    slab = c * NS + s

    pltpu.sync_copy(pk_hbm.at[slab, 0], pk_t)
    pltpu.sync_copy(wgt_hbm.at[slab, 0], wgt_t)

    def _zero_row(r, carry):
        for f in range(D // 16):
            gbuf[0, r, pl.ds(f * 16, 16)] = jnp.zeros((16,), jnp.float32)
        return carry
    lax.fori_loop(0, W, _zero_row, 0)
    base = s * ROW_STRIDE
    for t in range(ROW_CHUNK // W):
        pltpu.sync_copy(gbuf.at[0], agg.at[pl.ds(base + t * W, W)])
    plsc.subcore_barrier()

    def _unpack(j, b):
        for v in range(W // 16):
            pk = pk_t[pl.ds(j * W + v * 16, 16)]
            idxb[b, pl.ds(v * 16, 16)] = lax.shift_right_logical(pk, 14)
            idxb[NB + b, pl.ds(v * 16, 16)] = lax.bitwise_and(pk, PACK - 1)

    def _gather(j, b):
        _unpack(j, b)
        pltpu.async_copy(h_hbm.at[idxb.at[b]], gbuf.at[b], gsem.at[b])

    def _gather_wait(b):
        pltpu.make_async_copy(
            h_hbm.at[idxb.at[b]], gbuf.at[b], gsem.at[b]).wait()

    def _scatter(b):
        pltpu.async_copy(gbuf.at[b], agg.at[idxb.at[NB + b]], ssem.at[b],
                         add=True)

    def _scatter_wait(b):
        pltpu.make_async_copy(gbuf.at[b], agg.at[idxb.at[NB + b]],
                              ssem.at[b]).wait()

    _gather(0, 0)
    _gather(1, 1)

    mask_hi = jnp.full((16,), 0xFFFF0000, jnp.uint32)
    shift16 = jnp.full((16,), 16, jnp.uint32)

    def _wexpand(vec_i32):
        u32 = plsc.bitcast(vec_i32, jnp.uint32)
        lo = plsc.bitcast(lax.shift_left(u32, shift16), jnp.float32)
        hi = plsc.bitcast(lax.bitwise_and(u32, mask_hi), jnp.float32)
        return lo, hi

    def _group(g, carry):
        for b in range(NB):
            j = g * NB + b
            bf = (b + 2) % NB

            _gather_wait(b)

            if b == 0:
                @pl.when(g >= 1)
                def _():
                    _scatter_wait(bf)
                _gather(j + 2, bf)
            else:
                @pl.when(g < NG - 1)
                def _():
                    _scatter_wait(bf)
                    _gather(j + 2, bf)

            def _scale_blk(k, carry2):
                for pair in range(2):
                    e0 = k * 4 + pair * 2
                    pidx = jnp.full((16,), 0, jnp.int32) + (j * (W // 2)
                                                            + e0 // 2)
                    w_even, w_odd = _wexpand(plsc.load_gather(wgt_t, [pidx]))
                    for e, wv in ((e0, w_even), (e0 + 1, w_odd)):
                        for f in range(D // 16):
                            sl = pl.ds(f * 16, 16)
                            gbuf[b, e, sl] = gbuf[b, e, sl] * wv
                return carry2
            lax.fori_loop(0, W // 4, _scale_blk, 0)

            _scatter(b)
        return carry
    lax.fori_loop(0, NG, _group, 0)

    for b in range(NB):
        _scatter_wait(b)

    plsc.subcore_barrier()
    pltpu.sync_copy(agg.at[pl.ds(base, ROW_CHUNK)],
                    out_hbm.at[c, pl.ds(base, ROW_CHUNK)])


def kernel(x, edge_index, edge_weight, W1, b1, W2, b2):
    h = _mlp(x, W1, b1, W2, b2)

    pad = E_PAD - E
    fill = (jnp.arange(pad, dtype=jnp.int32) * 997) % N
    col = jnp.concatenate([edge_index[1], fill])
    row = jnp.concatenate([edge_index[0], fill])
    packed = (col * PACK + row).reshape(NTILES, 1, EPT)
    wf = jnp.concatenate([edge_weight, jnp.zeros((pad,), jnp.float32)])
    wu = lax.bitcast_convert_type(wf, jnp.uint32)
    wr = (wu + 0x7FFF + ((wu >> 16) & 1)) >> 16
    wpk = lax.bitcast_convert_type(
        wr[0::2] | (wr[1::2] << 16), jnp.int32).reshape(NTILES, 1, EPT // 2)

    for _ in range(K):
        parts = _propagate(h, packed, wpk)
        h = _combine(parts, h)
    return h
